# baseline (speedup 1.0000x reference)
import functools

import jax
import jax.numpy as jnp
from jax import lax
from jax.experimental import pallas as pl
from jax.experimental.pallas import tpu as pltpu
from jax.experimental.pallas import tpu_sc as plsc

N_STATES = 1000000
OUTPUT_DIM = 16
BATCH = 16384

_info = plsc.get_sparse_core_info()
_NC, _NS = _info.num_cores, _info.num_subcores
_NW = _NC * _NS
_B_PER_W = BATCH // _NW
_UNROLL = 16
_NSTEP = _B_PER_W // _UNROLL

_mesh = plsc.VectorSubcoreMesh(core_axis_name="c", subcore_axis_name="s")


@functools.partial(
    pl.kernel,
    mesh=_mesh,
    out_type=jax.ShapeDtypeStruct((BATCH, OUTPUT_DIM), jnp.float32),
    scratch_types=[
        pltpu.VMEM((_B_PER_W,), jnp.int32),
        pltpu.SemaphoreType.DMA,
    ],
)
def _gather_kernel(table_hbm, idx_hbm, out_hbm, idx_v, sem):
    wid = lax.axis_index("s") * _NC + lax.axis_index("c")
    base = wid * _B_PER_W
    pltpu.sync_copy(idx_hbm.at[pl.ds(base, _B_PER_W)], idx_v)

    def _start(g, _):
        vec = idx_v[pl.ds(g * _UNROLL, _UNROLL)]
        for b in range(_UNROLL):
            i = g * _UNROLL + b
            r = vec[b]
            pltpu.make_async_copy(
                table_hbm.at[pl.ds(r, 1), :],
                out_hbm.at[pl.ds(base + i, 1), :],
                sem,
            ).start()
        return ()

    lax.fori_loop(0, _NSTEP, _start, ())

    def _drain(g, _):
        for b in range(_UNROLL):
            i = g * _UNROLL + b
            pltpu.make_async_copy(
                table_hbm.at[pl.ds(0, 1), :],
                out_hbm.at[pl.ds(base + i, 1), :],
                sem,
            ).wait()
        return ()

    lax.fori_loop(0, _NSTEP, _drain, ())


def kernel(preprocessed_states, table):
    idx = jnp.reshape(preprocessed_states, (BATCH,)).astype(jnp.int32)
    return _gather_kernel(table, idx)


# speedup vs baseline: 1.8753x; 1.8753x over previous
import functools

import jax
import jax.numpy as jnp
from jax import lax
from jax.experimental import pallas as pl
from jax.experimental.pallas import tpu as pltpu
from jax.experimental.pallas import tpu_sc as plsc

N_STATES = 1000000
OUTPUT_DIM = 16
BATCH = 16384

_info = plsc.get_sparse_core_info()
_NC, _NS = _info.num_cores, _info.num_subcores
_NW = _NC * _NS
_B_PER_W = BATCH // _NW
_UNROLL = 16
_NSTEP = _B_PER_W // _UNROLL

_mesh = plsc.VectorSubcoreMesh(core_axis_name="c", subcore_axis_name="s")


@functools.partial(
    pl.kernel,
    mesh=_mesh,
    out_type=jax.ShapeDtypeStruct((BATCH, OUTPUT_DIM), jnp.float32),
    scratch_types=[
        pltpu.VMEM((_B_PER_W,), jnp.int32),
        pltpu.VMEM((_B_PER_W, OUTPUT_DIM), jnp.float32),
        pltpu.SemaphoreType.DMA,
    ],
)
def _gather_kernel(table_hbm, idx_hbm, out_hbm, idx_v, rows_v, sem):
    wid = lax.axis_index("s") * _NC + lax.axis_index("c")
    base = wid * _B_PER_W
    pltpu.sync_copy(idx_hbm.at[pl.ds(base, _B_PER_W)], idx_v)

    def _start(g, _):
        vec = idx_v[pl.ds(g * _UNROLL, _UNROLL)]
        for b in range(_UNROLL):
            i = g * _UNROLL + b
            r = vec[b]
            pltpu.make_async_copy(
                table_hbm.at[pl.ds(r, 1), :],
                rows_v.at[pl.ds(i, 1), :],
                sem,
            ).start()
        return ()

    lax.fori_loop(0, _NSTEP, _start, ())

    pltpu.make_async_copy(
        table_hbm.at[pl.ds(0, _B_PER_W), :], rows_v, sem
    ).wait()

    pltpu.sync_copy(rows_v, out_hbm.at[pl.ds(base, _B_PER_W), :])


def kernel(preprocessed_states, table):
    idx = jnp.reshape(preprocessed_states, (BATCH,)).astype(jnp.int32)
    return _gather_kernel(table, idx)
